# baseline (speedup 1.0000x reference)
import dataclasses

import jax
import jax.numpy as jnp
from jax import lax
from jax.experimental import pallas as pl
from jax.experimental.pallas import tpu as pltpu
from jax.experimental.pallas import tpu_sc as plsc

_N = 8192
_D = 2048
_ALPHA = 1.0
_L = 16
_RB = 8
_NCH = _D // _L


def _full(val):
    return jnp.full((_L,), val, dtype=jnp.float32)


def _sc_kernel_fn(t_hbm, m_hbm, d_hbm, o_hbm, d_v, sem):
    pltpu.async_copy(d_hbm, d_v, sem).wait()

    def gram_body(k, accs):
        a11, a22, a33, a12, a13, a23 = accs
        off = k * _L
        v1 = d_v[0, pl.ds(off, _L)]
        v2 = d_v[1, pl.ds(off, _L)]
        v3 = d_v[2, pl.ds(off, _L)]
        return (a11 + v1 * v1, a22 + v2 * v2, a33 + v3 * v3,
                a12 + v1 * v2, a13 + v1 * v3, a23 + v2 * v3)

    z = jnp.zeros((_L,), jnp.float32)
    a11, a22, a33, a12, a13, a23 = lax.fori_loop(
        0, _NCH, gram_body, (z, z, z, z, z, z))
    ones = _full(1.0)
    inv1 = ones / _full(jnp.sum(a11))
    inv2 = ones / _full(jnp.sum(a22))
    inv3 = ones / _full(jnp.sum(a33))
    g12 = _full(jnp.sum(a12))
    g13 = _full(jnp.sum(a13))
    g23 = _full(jnp.sum(a23))
    alphav = _full(_ALPHA)

    def block_body(t_v, m_v, o_v):
        mvec = m_v[0, pl.ds(0, _L)]
        for r in range(_RB):
            mval = mvec[r]

            @pl.when(mval > 0.5)
            def _compute():
                def dot_body(k, accs):
                    c1, c2, c3 = accs
                    off = k * _L
                    tv = t_v[r, pl.ds(off, _L)]
                    return (c1 + tv * d_v[0, pl.ds(off, _L)],
                            c2 + tv * d_v[1, pl.ds(off, _L)],
                            c3 + tv * d_v[2, pl.ds(off, _L)])

                c1a, c2a, c3a = lax.fori_loop(0, _NCH, dot_body, (z, z, z))
                c1 = _full(jnp.sum(c1a))
                c2 = _full(jnp.sum(c2a))
                c3 = _full(jnp.sum(c3a))
                a1 = c1 * inv1
                a2 = (c2 - a1 * g12) * inv2
                a3 = (c3 - a1 * g13 - a2 * g23) * inv3
                am3 = a3 - alphav

                @pl.loop(0, _NCH)
                def _tail(k):
                    off = k * _L
                    o_v[r, pl.ds(off, _L)] = (
                        ((t_v[r, pl.ds(off, _L)]
                          - a1 * d_v[0, pl.ds(off, _L)])
                         - a2 * d_v[1, pl.ds(off, _L)])
                        - am3 * d_v[2, pl.ds(off, _L)])

            @pl.when(mval <= 0.5)
            def _copy():
                @pl.loop(0, _NCH)
                def _cp(k):
                    off = k * _L
                    o_v[r, pl.ds(off, _L)] = t_v[r, pl.ds(off, _L)]

    pltpu.emit_pipeline(
        block_body,
        grid=(_N // _RB,),
        in_specs=[
            pl.BlockSpec((_RB, _D), lambda i: (i, 0)),
            pl.BlockSpec((1, _L), lambda i: (i, 0)),
        ],
        out_specs=[pl.BlockSpec((_RB, _D), lambda i: (i, 0))],
        core_axis_name=("c", "s"),
        dimension_semantics=(pltpu.PARALLEL,),
    )(t_hbm, m_hbm, o_hbm)


def kernel(t_embs, token_mask, delta_front, delta_side, delta_back):
    mf = token_mask.astype(jnp.float32).reshape(_N // _RB, _RB)
    m2 = jnp.tile(mf, (1, _L // _RB))
    d = jnp.stack([delta_front, delta_side, delta_back], axis=0)

    mesh = plsc.VectorSubcoreMesh(core_axis_name="c", subcore_axis_name="s")
    cp = pltpu.CompilerParams()
    if "needs_layout_passes" in pltpu.CompilerParams.__dataclass_fields__:
        cp = dataclasses.replace(cp, needs_layout_passes=False)
    f = pl.kernel(
        _sc_kernel_fn,
        out_type=jax.ShapeDtypeStruct((_N, _D), jnp.float32),
        mesh=mesh,
        compiler_params=cp,
        scratch_types=[
            pltpu.VMEM((3, _D), jnp.float32),
            pltpu.SemaphoreType.DMA,
        ],
    )
    return f(t_embs, m2, d)


# speedup vs baseline: 1.0755x; 1.0755x over previous
import dataclasses

import jax
import jax.numpy as jnp
from jax import lax
from jax.experimental import pallas as pl
from jax.experimental.pallas import tpu as pltpu
from jax.experimental.pallas import tpu_sc as plsc

_N = 8192
_D = 2048
_ALPHA = 1.0
_L = 16
_RB = 8
_NCH = _D // _L


def _full(val):
    return jnp.full((_L,), val, dtype=jnp.float32)


def _sc_kernel_fn(t_hbm, m_hbm, d_hbm, o_hbm, d_v, sem):
    pltpu.async_copy(d_hbm, d_v, sem).wait()

    def gram_body(k, accs):
        a11, a22, a33, a12, a13, a23 = accs
        off = k * _L
        v1 = d_v[0, pl.ds(off, _L)]
        v2 = d_v[1, pl.ds(off, _L)]
        v3 = d_v[2, pl.ds(off, _L)]
        return (a11 + v1 * v1, a22 + v2 * v2, a33 + v3 * v3,
                a12 + v1 * v2, a13 + v1 * v3, a23 + v2 * v3)

    z = jnp.zeros((_L,), jnp.float32)
    a11, a22, a33, a12, a13, a23 = lax.fori_loop(
        0, _NCH, gram_body, (z, z, z, z, z, z))
    ones = _full(1.0)
    inv1 = ones / _full(jnp.sum(a11))
    inv2 = ones / _full(jnp.sum(a22))
    inv3 = ones / _full(jnp.sum(a33))
    g12 = _full(jnp.sum(a12))
    g13 = _full(jnp.sum(a13))
    g23 = _full(jnp.sum(a23))
    alphav = _full(_ALPHA)

    def block_body(t_v, m_v, o_v):
        mvec = m_v[0, pl.ds(0, _L)]
        for r in range(_RB):
            mval = mvec[r]

            @pl.when(mval > 0.5)
            def _compute():
                def dot_body(k, accs):
                    c1, c2, c3 = accs
                    off = k * _L
                    tv = t_v[r, pl.ds(off, _L)]
                    return (c1 + tv * d_v[0, pl.ds(off, _L)],
                            c2 + tv * d_v[1, pl.ds(off, _L)],
                            c3 + tv * d_v[2, pl.ds(off, _L)])

                c1a, c2a, c3a = lax.fori_loop(0, _NCH, dot_body, (z, z, z),
                                              unroll=8)
                c1 = _full(jnp.sum(c1a))
                c2 = _full(jnp.sum(c2a))
                c3 = _full(jnp.sum(c3a))
                a1 = c1 * inv1
                a2 = (c2 - a1 * g12) * inv2
                a3 = (c3 - a1 * g13 - a2 * g23) * inv3
                am3 = a3 - alphav

                def tail_body(k, carry):
                    off = k * _L
                    o_v[r, pl.ds(off, _L)] = (
                        ((t_v[r, pl.ds(off, _L)]
                          - a1 * d_v[0, pl.ds(off, _L)])
                         - a2 * d_v[1, pl.ds(off, _L)])
                        - am3 * d_v[2, pl.ds(off, _L)])
                    return carry

                lax.fori_loop(0, _NCH, tail_body, 0, unroll=8)

            @pl.when(mval <= 0.5)
            def _copy():
                def copy_body(k, carry):
                    off = k * _L
                    o_v[r, pl.ds(off, _L)] = t_v[r, pl.ds(off, _L)]
                    return carry

                lax.fori_loop(0, _NCH, copy_body, 0, unroll=8)

    pltpu.emit_pipeline(
        block_body,
        grid=(_N // _RB,),
        in_specs=[
            pl.BlockSpec((_RB, _D), lambda i: (i, 0)),
            pl.BlockSpec((1, _L), lambda i: (i, 0)),
        ],
        out_specs=[pl.BlockSpec((_RB, _D), lambda i: (i, 0))],
        core_axis_name=("c", "s"),
        dimension_semantics=(pltpu.PARALLEL,),
    )(t_hbm, m_hbm, o_hbm)


def kernel(t_embs, token_mask, delta_front, delta_side, delta_back):
    mf = token_mask.astype(jnp.float32).reshape(_N // _RB, _RB)
    m2 = jnp.tile(mf, (1, _L // _RB))
    d = jnp.stack([delta_front, delta_side, delta_back], axis=0)

    mesh = plsc.VectorSubcoreMesh(core_axis_name="c", subcore_axis_name="s")
    cp = pltpu.CompilerParams()
    if "needs_layout_passes" in pltpu.CompilerParams.__dataclass_fields__:
        cp = dataclasses.replace(cp, needs_layout_passes=False)
    f = pl.kernel(
        _sc_kernel_fn,
        out_type=jax.ShapeDtypeStruct((_N, _D), jnp.float32),
        mesh=mesh,
        compiler_params=cp,
        scratch_types=[
            pltpu.VMEM((3, _D), jnp.float32),
            pltpu.SemaphoreType.DMA,
        ],
    )
    return f(t_embs, m2, d)


# speedup vs baseline: 1.2142x; 1.1289x over previous
import dataclasses

import jax
import jax.numpy as jnp
from jax import lax
from jax.experimental import pallas as pl
from jax.experimental.pallas import tpu as pltpu
from jax.experimental.pallas import tpu_sc as plsc

_N = 8192
_D = 2048
_ALPHA = 1.0
_L = 16
_NW = 32
_RW = _N // _NW
_RB = 8
_NBLK = _RW // _RB
_NS = 4
_NCH = _D // _L


def _full(val):
    return jnp.full((_L,), val, dtype=jnp.float32)


def _sc_kernel_fn(t_hbm, m_hbm, d_hbm, o_hbm, d_v, m_v, buf, insem, outsem):
    wid = lax.axis_index("s") * 2 + lax.axis_index("c")
    base = wid * _RW

    pltpu.sync_copy(d_hbm, d_v)
    pltpu.sync_copy(m_hbm.at[pl.ds(base, _RW)], m_v.at[pl.ds(0, _RW)])

    def gram_body(k, accs):
        a11, a22, a33, a12, a13, a23 = accs
        off = k * _L
        v1 = d_v[0, pl.ds(off, _L)]
        v2 = d_v[1, pl.ds(off, _L)]
        v3 = d_v[2, pl.ds(off, _L)]
        return (a11 + v1 * v1, a22 + v2 * v2, a33 + v3 * v3,
                a12 + v1 * v2, a13 + v1 * v3, a23 + v2 * v3)

    z = jnp.zeros((_L,), jnp.float32)
    a11, a22, a33, a12, a13, a23 = lax.fori_loop(
        0, _NCH, gram_body, (z, z, z, z, z, z), unroll=4)
    ones = _full(1.0)
    inv1 = ones / _full(jnp.sum(a11))
    inv2 = ones / _full(jnp.sum(a22))
    inv3 = ones / _full(jnp.sum(a33))
    g12 = _full(jnp.sum(a12))
    g13 = _full(jnp.sum(a13))
    g23 = _full(jnp.sum(a23))
    alphav = _full(_ALPHA)

    def in_copy(b, s):
        return pltpu.make_async_copy(
            t_hbm.at[pl.ds(base + b * _RB, _RB)], buf.at[s], insem.at[s])

    def out_copy(b, s):
        return pltpu.make_async_copy(
            buf.at[s], o_hbm.at[pl.ds(base + b * _RB, _RB)], outsem.at[s])

    def process_block(b, s):
        mvec = m_v[pl.ds(b * _RB, _L)]
        for r in range(_RB):
            mval = mvec[r]

            @pl.when(mval > 0.5)
            def _compute():
                def dot_body(k, accs):
                    c1, c2, c3 = accs
                    off = k * _L
                    tv = buf[s, r, pl.ds(off, _L)]
                    return (c1 + tv * d_v[0, pl.ds(off, _L)],
                            c2 + tv * d_v[1, pl.ds(off, _L)],
                            c3 + tv * d_v[2, pl.ds(off, _L)])

                c1a, c2a, c3a = lax.fori_loop(0, _NCH, dot_body, (z, z, z),
                                              unroll=8)
                c1 = _full(jnp.sum(c1a))
                c2 = _full(jnp.sum(c2a))
                c3 = _full(jnp.sum(c3a))
                a1 = c1 * inv1
                a2 = (c2 - a1 * g12) * inv2
                a3 = (c3 - a1 * g13 - a2 * g23) * inv3
                am3 = a3 - alphav

                def tail_body(k, carry):
                    off = k * _L
                    buf[s, r, pl.ds(off, _L)] = (
                        ((buf[s, r, pl.ds(off, _L)]
                          - a1 * d_v[0, pl.ds(off, _L)])
                         - a2 * d_v[1, pl.ds(off, _L)])
                        - am3 * d_v[2, pl.ds(off, _L)])
                    return carry

                lax.fori_loop(0, _NCH, tail_body, 0, unroll=8)

    for s in range(_NS):
        in_copy(s, s).start()

    @pl.loop(0, _NBLK, step=_NS)
    def _(i):
        for s in range(_NS):
            b = i + s
            in_copy(b, s).wait()
            process_block(b, s)
            out_copy(b, s).start()
            nxt = b + _NS

            @pl.when(nxt < _NBLK)
            def _():
                out_copy(b, s).wait()
                in_copy(nxt, s).start()

    for s in range(_NS):
        out_copy(_NBLK - _NS + s, s).wait()


def kernel(t_embs, token_mask, delta_front, delta_side, delta_back):
    mf = token_mask.astype(jnp.float32)
    d = jnp.stack([delta_front, delta_side, delta_back], axis=0)

    mesh = plsc.VectorSubcoreMesh(core_axis_name="c", subcore_axis_name="s")
    cp = pltpu.CompilerParams()
    if "needs_layout_passes" in pltpu.CompilerParams.__dataclass_fields__:
        cp = dataclasses.replace(cp, needs_layout_passes=False)
    f = pl.kernel(
        _sc_kernel_fn,
        out_type=jax.ShapeDtypeStruct((_N, _D), jnp.float32),
        mesh=mesh,
        compiler_params=cp,
        scratch_types=[
            pltpu.VMEM((3, _D), jnp.float32),
            pltpu.VMEM((_RW + _L,), jnp.float32),
            pltpu.VMEM((_NS, _RB, _D), jnp.float32),
            pltpu.SemaphoreType.DMA((_NS,)),
            pltpu.SemaphoreType.DMA((_NS,)),
        ],
    )
    return f(t_embs, mf, d)


# speedup vs baseline: 5.6215x; 4.6300x over previous
import jax
import jax.numpy as jnp
from jax.experimental import pallas as pl
from jax.experimental.pallas import tpu as pltpu

_N = 8192
_D = 2048
_ALPHA = 1.0
_BR = 512


def _tc_body(t_ref, m_ref, d_ref, o_ref):
    d = d_ref[...]
    t = t_ref[...]
    m = m_ref[...]

    d1 = d[0:1, :]
    d2 = d[1:2, :]
    d3 = d[2:3, :]
    n1 = jnp.sum(d1 * d1)
    n2 = jnp.sum(d2 * d2)
    n3 = jnp.sum(d3 * d3)
    g12 = jnp.sum(d1 * d2)
    g13 = jnp.sum(d1 * d3)
    g23 = jnp.sum(d2 * d3)

    c1 = jnp.sum(t * d1, axis=1, keepdims=True)
    c2 = jnp.sum(t * d2, axis=1, keepdims=True)
    c3 = jnp.sum(t * d3, axis=1, keepdims=True)

    a1 = c1 / n1
    a2 = (c2 - a1 * g12) / n2
    a3 = (c3 - a1 * g13 - a2 * g23) / n3

    comb = a1 * d1 + a2 * d2 + (a3 - _ALPHA) * d3
    o_ref[...] = t - m * comb


def kernel(t_embs, token_mask, delta_front, delta_side, delta_back):
    m = token_mask.astype(jnp.float32).reshape(_N, 1)
    d = jnp.stack([delta_front, delta_side, delta_back], axis=0)
    grid = (_N // _BR,)
    return pl.pallas_call(
        _tc_body,
        grid=grid,
        in_specs=[
            pl.BlockSpec((_BR, _D), lambda i: (i, 0)),
            pl.BlockSpec((_BR, 1), lambda i: (i, 0)),
            pl.BlockSpec((3, _D), lambda i: (0, 0)),
        ],
        out_specs=pl.BlockSpec((_BR, _D), lambda i: (i, 0)),
        out_shape=jax.ShapeDtypeStruct((_N, _D), jnp.float32),
    )(t_embs, m, d)
